# Initial kernel scaffold; baseline (speedup 1.0000x reference)
import jax
import jax.numpy as jnp
from jax.experimental import pallas as pl


def kernel(x, edge_index, W1, b1, W2, b2):
    raise NotImplementedError("write your pallas kernel here")


# speedup vs baseline: 26.9500x; 26.9500x over previous
import functools

import jax
import jax.numpy as jnp
from jax import lax
from jax.experimental import pallas as pl
from jax.experimental.pallas import tpu as pltpu
from jax.experimental.pallas import tpu_sc as plsc

N = 10000
NA = 10240
DUMP = 10000
ZROW = 10001
E_PAD = 32 * 80 * 128
IDXROWS = E_PAD // 128
RPT = 80
NPT = NA // 16
D1 = 16
D2 = 48

_MESH = dict(core_axis_name="c", subcore_axis_name="s")
_SC_PARAMS = pltpu.CompilerParams(use_tc_tiling_on_sc=False)


def _zero_fill(ref, n_rows, row_words):
  zero16 = jnp.zeros((16,), jnp.float32)

  def body(i, _):
    for j in range(row_words // 16):
      ref[i, pl.ds(j * 16, 16)] = zero16
    return 0

  lax.fori_loop(0, n_rows, body, 0)


def _make_deg():
  mesh = plsc.VectorSubcoreMesh(**_MESH)

  @functools.partial(
      pl.kernel,
      out_type=jax.ShapeDtypeStruct((2, NA), jnp.float32),
      mesh=mesh,
      compiler_params=_SC_PARAMS,
      scratch_types=[
          pltpu.VMEM((RPT, 128), jnp.int32),
          pltpu.VMEM((128,), jnp.float32),
          pltpu.VMEM((NPT,), jnp.float32),
          pltpu.VMEM_SHARED((NA,), jnp.float32),
      ],
  )
  def deg_kernel(dst_hbm, out_hbm, idx_v, ones_v, zero_v, acc):
    c = lax.axis_index("c")
    s = lax.axis_index("s")
    wid = c * 16 + s
    one16 = jnp.full((16,), 1.0, jnp.float32)
    zero16 = jnp.zeros((16,), jnp.float32)
    for i in range(8):
      ones_v[pl.ds(i * 16, 16)] = one16

    def zbody(i, _):
      zero_v[pl.ds(i * 16, 16)] = zero16
      return 0

    lax.fori_loop(0, NPT // 16, zbody, 0)
    pltpu.sync_copy(zero_v, acc.at[pl.ds(s * NPT, NPT)])
    pltpu.sync_copy(dst_hbm.at[pl.ds(wid * RPT, RPT)], idx_v)
    plsc.subcore_barrier()

    def body(j, _):
      pltpu.sync_copy(ones_v, acc.at[idx_v.at[j]], add=True)
      return 0

    lax.fori_loop(0, RPT, body, 0)
    plsc.subcore_barrier()
    pltpu.sync_copy(acc.at[pl.ds(s * NPT, NPT)],
                    out_hbm.at[c, pl.ds(s * NPT, NPT)])

  return deg_kernel


def _make_pass(d):
  mesh = plsc.VectorSubcoreMesh(**_MESH)

  @functools.partial(
      pl.kernel,
      out_type=jax.ShapeDtypeStruct((2, NA, d), jnp.float32),
      mesh=mesh,
      compiler_params=_SC_PARAMS,
      scratch_types=[
          pltpu.VMEM((RPT, 128), jnp.int32),
          pltpu.VMEM((RPT, 128), jnp.int32),
          pltpu.VMEM((128, d), jnp.float32),
          pltpu.VMEM((128, d), jnp.float32),
          pltpu.VMEM((128, d), jnp.float32),
          pltpu.VMEM_SHARED((NA, d), jnp.float32),
          pltpu.SemaphoreType.DMA,
          pltpu.SemaphoreType.DMA,
      ],
  )
  def pass_kernel(hs_hbm, src_hbm, dst_hbm, out_hbm,
                  sidx, didx, rows0, rows1, zrow, acc, sem0, sem1):
    c = lax.axis_index("c")
    s = lax.axis_index("s")
    wid = c * 16 + s
    rb = wid * RPT
    _zero_fill(zrow, 128, d)
    for k in range(NPT // 128):
      pltpu.sync_copy(zrow, acc.at[pl.ds(s * NPT + k * 128, 128)])
    pltpu.sync_copy(src_hbm.at[pl.ds(rb, RPT)], sidx)
    pltpu.sync_copy(dst_hbm.at[pl.ds(rb, RPT)], didx)
    plsc.subcore_barrier()

    pltpu.async_copy(hs_hbm.at[sidx.at[0]], rows0, sem0)

    def body(i, _):
      b0 = 2 * i
      b1 = 2 * i + 1
      pltpu.async_copy(hs_hbm.at[sidx.at[b1]], rows1, sem1)
      pltpu.make_async_copy(hs_hbm.at[sidx.at[b0]], rows0, sem0).wait()
      pltpu.sync_copy(rows0, acc.at[didx.at[b0]], add=True)

      @pl.when(i < RPT // 2 - 1)
      def _():
        pltpu.async_copy(hs_hbm.at[sidx.at[b0 + 2]], rows0, sem0)

      pltpu.make_async_copy(hs_hbm.at[sidx.at[b1]], rows1, sem1).wait()
      pltpu.sync_copy(rows1, acc.at[didx.at[b1]], add=True)
      return 0

    lax.fori_loop(0, RPT // 2, body, 0)
    plsc.subcore_barrier()
    pltpu.sync_copy(acc.at[pl.ds(s * NPT, NPT)],
                    out_hbm.at[c, pl.ds(s * NPT, NPT)])

  return pass_kernel


_deg_call = _make_deg()
_pass16 = _make_pass(D1)
_pass48 = _make_pass(D2)


def _stage_a_body(x_ref, w_ref, d_ref, o_ref):
  o_ref[:, :] = jnp.dot(x_ref[:, :], w_ref[:, :],
                        preferred_element_type=jnp.float32) * d_ref[:, :]


def _stage_a(x_pad, w1, dinv16):
  grid = (NA // 1024,)
  return pl.pallas_call(
      _stage_a_body,
      grid=grid,
      in_specs=[
          pl.BlockSpec((1024, 128), lambda i: (i, 0)),
          pl.BlockSpec((128, D1), lambda i: (0, 0)),
          pl.BlockSpec((1024, D1), lambda i: (i, 0)),
      ],
      out_specs=pl.BlockSpec((1024, D1), lambda i: (i, 0)),
      out_shape=jax.ShapeDtypeStruct((NA, D1), jnp.float32),
  )(x_pad, w1, dinv16)


def _stage_b_body(p_ref, h_ref, d16_ref, d48_ref, b1_ref, w2_ref, o_ref):
  t = (p_ref[0] + p_ref[1] + h_ref[:, :]) * d16_ref[:, :] + b1_ref[:, :]
  t = jnp.maximum(t, 0.0)
  o_ref[:, :] = jnp.dot(t, w2_ref[:, :],
                        preferred_element_type=jnp.float32) * d48_ref[:, :]


def _stage_b(p1, hs1, dinv16, dinv48, b1r, w2p):
  grid = (NA // 1024,)
  return pl.pallas_call(
      _stage_b_body,
      grid=grid,
      in_specs=[
          pl.BlockSpec((2, 1024, D1), lambda i: (0, i, 0)),
          pl.BlockSpec((1024, D1), lambda i: (i, 0)),
          pl.BlockSpec((1024, D1), lambda i: (i, 0)),
          pl.BlockSpec((1024, D2), lambda i: (i, 0)),
          pl.BlockSpec((1, D1), lambda i: (0, 0)),
          pl.BlockSpec((D1, D2), lambda i: (0, 0)),
      ],
      out_specs=pl.BlockSpec((1024, D2), lambda i: (i, 0)),
      out_shape=jax.ShapeDtypeStruct((NA, D2), jnp.float32),
  )(p1, hs1, dinv16, dinv48, b1r, w2p)


def _stage_c_body(p_ref, h_ref, d48_ref, b2_ref, o_ref):
  logits = (p_ref[0] + p_ref[1] + h_ref[:, :]) * d48_ref[:, :] + b2_ref[:, :]
  m = jnp.max(logits, axis=1, keepdims=True)
  e = jnp.exp(logits - m)
  ssum = jnp.sum(e, axis=1, keepdims=True)
  o_ref[:, :] = logits - m - jnp.log(ssum)


def _stage_c(p2, hs2, dinv48, b2p):
  grid = (NA // 1024,)
  return pl.pallas_call(
      _stage_c_body,
      grid=grid,
      in_specs=[
          pl.BlockSpec((2, 1024, D2), lambda i: (0, i, 0)),
          pl.BlockSpec((1024, D2), lambda i: (i, 0)),
          pl.BlockSpec((1024, D2), lambda i: (i, 0)),
          pl.BlockSpec((1, D2), lambda i: (0, 0)),
      ],
      out_specs=pl.BlockSpec((1024, D2), lambda i: (i, 0)),
      out_shape=jax.ShapeDtypeStruct((NA, D2), jnp.float32),
  )(p2, hs2, dinv48, b2p)


def kernel(x, edge_index, W1, b1, W2, b2):
  src = edge_index[0].astype(jnp.int32)
  dst = edge_index[1].astype(jnp.int32)
  e = src.shape[0]
  pad = E_PAD - e
  src2d = jnp.concatenate(
      [src, jnp.full((pad,), ZROW, jnp.int32)]).reshape(IDXROWS, 128)
  dst2d = jnp.concatenate(
      [dst, jnp.full((pad,), DUMP, jnp.int32)]).reshape(IDXROWS, 128)
  x_pad = jnp.pad(x, ((0, NA - N), (0, 0)))

  deg_p = _deg_call(dst2d)
  deg = deg_p[0] + deg_p[1] + (jnp.arange(NA) < N).astype(jnp.float32)
  dinv = jnp.where(deg > 0, lax.rsqrt(deg), 0.0)
  dinv16 = jnp.broadcast_to(dinv[:, None], (NA, D1))
  dinv48 = jnp.broadcast_to(dinv[:, None], (NA, D2))

  hs1 = _stage_a(x_pad, W1, dinv16)
  p1 = _pass16(hs1, src2d, dst2d)

  w2p = jnp.pad(W2, ((0, 0), (0, D2 - W2.shape[1])))
  b1r = b1.reshape(1, D1)
  b2p = jnp.concatenate(
      [b2, jnp.full((D2 - b2.shape[0],), -1e30, jnp.float32)]).reshape(1, D2)

  hs2 = _stage_b(p1, hs1, dinv16, dinv48, b1r, w2p)
  p2 = _pass48(hs2, src2d, dst2d)
  out = _stage_c(p2, hs2, dinv48, b2p)
  return out[:N, :40]


# speedup vs baseline: 46.2373x; 1.7157x over previous
import functools

import jax
import jax.numpy as jnp
from jax import lax
from jax.experimental import pallas as pl
from jax.experimental.pallas import tpu as pltpu
from jax.experimental.pallas import tpu_sc as plsc

N = 10000
NA = 10240
DUMP = 10000
ZROW = 10001
E_PAD = 32 * 80 * 128
IDXROWS = E_PAD // 128
RPT = 80
NPT = NA // 16
D1 = 16
D2 = 40
RBLK = 1024

_MESH = dict(core_axis_name="c", subcore_axis_name="s")
_SC_PARAMS = pltpu.CompilerParams(use_tc_tiling_on_sc=False)


def _make_deg():
  mesh = plsc.VectorSubcoreMesh(**_MESH)

  @functools.partial(
      pl.kernel,
      out_type=jax.ShapeDtypeStruct((2, NA), jnp.float32),
      mesh=mesh,
      compiler_params=_SC_PARAMS,
      scratch_types=[
          pltpu.VMEM((RPT, 128), jnp.int32),
          pltpu.VMEM((128,), jnp.float32),
          pltpu.VMEM((NPT,), jnp.float32),
          pltpu.VMEM_SHARED((NA,), jnp.float32),
      ],
  )
  def deg_kernel(dst_hbm, out_hbm, idx_v, ones_v, zero_v, acc):
    c = lax.axis_index("c")
    s = lax.axis_index("s")
    wid = c * 16 + s
    one16 = jnp.full((16,), 1.0, jnp.float32)
    zero16 = jnp.zeros((16,), jnp.float32)
    for i in range(8):
      ones_v[pl.ds(i * 16, 16)] = one16

    def zbody(i, _):
      zero_v[pl.ds(i * 16, 16)] = zero16
      return 0

    lax.fori_loop(0, NPT // 16, zbody, 0)
    pltpu.sync_copy(zero_v, acc.at[pl.ds(s * NPT, NPT)])
    pltpu.sync_copy(dst_hbm.at[pl.ds(wid * RPT, RPT)], idx_v)
    plsc.subcore_barrier()

    def body(j, _):
      pltpu.sync_copy(ones_v, acc.at[idx_v.at[j]], add=True)
      return 0

    lax.fori_loop(0, RPT, body, 0)
    plsc.subcore_barrier()
    pltpu.sync_copy(acc.at[pl.ds(s * NPT, NPT)],
                    out_hbm.at[c, pl.ds(s * NPT, NPT)])

  return deg_kernel


def _make_pass(d):
  mesh = plsc.VectorSubcoreMesh(**_MESH)

  @functools.partial(
      pl.kernel,
      out_type=jax.ShapeDtypeStruct((2, NA, d), jnp.float32),
      mesh=mesh,
      compiler_params=_SC_PARAMS,
      scratch_types=[
          pltpu.VMEM((RPT, 128), jnp.int32),
          pltpu.VMEM((RPT, 128), jnp.int32),
          pltpu.VMEM((128, d), jnp.float32),
          pltpu.VMEM((128, d), jnp.float32),
          pltpu.VMEM_SHARED((NA, d), jnp.float32),
          pltpu.VMEM_SHARED((NA, d), jnp.float32),
          pltpu.SemaphoreType.DMA,
          pltpu.SemaphoreType.DMA,
      ],
  )
  def pass_kernel(hs_hbm, src_hbm, dst_hbm, zz_hbm, out_hbm,
                  sidx, didx, rows0, rows1, table, acc, sem0, sem1):
    c = lax.axis_index("c")
    s = lax.axis_index("s")
    wid = c * 16 + s
    rb = wid * RPT
    pltpu.sync_copy(hs_hbm.at[pl.ds(s * NPT, NPT)],
                    table.at[pl.ds(s * NPT, NPT)])
    pltpu.sync_copy(zz_hbm, acc.at[pl.ds(s * NPT, NPT)])
    pltpu.sync_copy(src_hbm.at[pl.ds(rb, RPT)], sidx)
    pltpu.sync_copy(dst_hbm.at[pl.ds(rb, RPT)], didx)
    plsc.subcore_barrier()

    pltpu.async_copy(table.at[sidx.at[0]], rows0, sem0)

    def body(i, _):
      b0 = 2 * i
      b1 = 2 * i + 1
      pltpu.async_copy(table.at[sidx.at[b1]], rows1, sem1)
      pltpu.make_async_copy(table.at[sidx.at[b0]], rows0, sem0).wait()
      pltpu.sync_copy(rows0, acc.at[didx.at[b0]], add=True)

      @pl.when(i < RPT // 2 - 1)
      def _():
        pltpu.async_copy(table.at[sidx.at[b0 + 2]], rows0, sem0)

      pltpu.make_async_copy(table.at[sidx.at[b1]], rows1, sem1).wait()
      pltpu.sync_copy(rows1, acc.at[didx.at[b1]], add=True)
      return 0

    lax.fori_loop(0, RPT // 2, body, 0)
    plsc.subcore_barrier()
    pltpu.sync_copy(acc.at[pl.ds(s * NPT, NPT)],
                    out_hbm.at[c, pl.ds(s * NPT, NPT)])

  return pass_kernel


_deg_call = _make_deg()
_pass16 = _make_pass(D1)
_pass40 = _make_pass(D2)


def _row_mask(i_blk):
  rows = (i_blk * RBLK
          + lax.broadcasted_iota(jnp.int32, (RBLK, 1), dimension=0))
  return rows < N


def _stage_a_body(x_ref, w_ref, d_ref, o_ref):
  h = jnp.dot(x_ref[:, :], w_ref[:, :],
              preferred_element_type=jnp.float32) * d_ref[:, :]
  o_ref[:, :] = jnp.where(_row_mask(pl.program_id(0)), h, 0.0)


def _stage_a(x, w1, dinv16):
  grid = (NA // RBLK,)
  return pl.pallas_call(
      _stage_a_body,
      grid=grid,
      in_specs=[
          pl.BlockSpec((RBLK, 128), lambda i: (i, 0)),
          pl.BlockSpec((128, D1), lambda i: (0, 0)),
          pl.BlockSpec((RBLK, D1), lambda i: (i, 0)),
      ],
      out_specs=pl.BlockSpec((RBLK, D1), lambda i: (i, 0)),
      out_shape=jax.ShapeDtypeStruct((NA, D1), jnp.float32),
  )(x, w1, dinv16)


def _stage_b_body(p_ref, h_ref, d16_ref, d40_ref, b1_ref, w2_ref, o_ref):
  t = (p_ref[0] + p_ref[1] + h_ref[:, :]) * d16_ref[:, :] + b1_ref[:, :]
  t = jnp.maximum(t, 0.0)
  h2 = jnp.dot(t, w2_ref[:, :],
               preferred_element_type=jnp.float32) * d40_ref[:, :]
  o_ref[:, :] = jnp.where(_row_mask(pl.program_id(0)), h2, 0.0)


def _stage_b(p1, hs1, dinv16, dinv40, b1r, w2):
  grid = (NA // RBLK,)
  return pl.pallas_call(
      _stage_b_body,
      grid=grid,
      in_specs=[
          pl.BlockSpec((2, RBLK, D1), lambda i: (0, i, 0)),
          pl.BlockSpec((RBLK, D1), lambda i: (i, 0)),
          pl.BlockSpec((RBLK, D1), lambda i: (i, 0)),
          pl.BlockSpec((RBLK, D2), lambda i: (i, 0)),
          pl.BlockSpec((1, D1), lambda i: (0, 0)),
          pl.BlockSpec((D1, D2), lambda i: (0, 0)),
      ],
      out_specs=pl.BlockSpec((RBLK, D2), lambda i: (i, 0)),
      out_shape=jax.ShapeDtypeStruct((NA, D2), jnp.float32),
  )(p1, hs1, dinv16, dinv40, b1r, w2)


def _stage_c_body(p_ref, h_ref, d40_ref, b2_ref, o_ref):
  logits = (p_ref[0] + p_ref[1] + h_ref[:, :]) * d40_ref[:, :] + b2_ref[:, :]
  m = jnp.max(logits, axis=1, keepdims=True)
  e = jnp.exp(logits - m)
  ssum = jnp.sum(e, axis=1, keepdims=True)
  o_ref[:, :] = logits - m - jnp.log(ssum)


def _stage_c(p2, hs2, dinv40, b2r):
  grid = (NA // RBLK,)
  return pl.pallas_call(
      _stage_c_body,
      grid=grid,
      in_specs=[
          pl.BlockSpec((2, RBLK, D2), lambda i: (0, i, 0)),
          pl.BlockSpec((RBLK, D2), lambda i: (i, 0)),
          pl.BlockSpec((RBLK, D2), lambda i: (i, 0)),
          pl.BlockSpec((1, D2), lambda i: (0, 0)),
      ],
      out_specs=pl.BlockSpec((RBLK, D2), lambda i: (i, 0)),
      out_shape=jax.ShapeDtypeStruct((NA, D2), jnp.float32),
  )(p2, hs2, dinv40, b2r)


def kernel(x, edge_index, W1, b1, W2, b2):
  src = edge_index[0].astype(jnp.int32)
  dst = edge_index[1].astype(jnp.int32)
  e = src.shape[0]
  pad = E_PAD - e
  src2d = jnp.concatenate(
      [src, jnp.full((pad,), ZROW, jnp.int32)]).reshape(IDXROWS, 128)
  dst2d = jnp.concatenate(
      [dst, jnp.full((pad,), DUMP, jnp.int32)]).reshape(IDXROWS, 128)

  deg_p = _deg_call(dst2d)
  deg = deg_p[0] + deg_p[1] + (jnp.arange(NA) < N).astype(jnp.float32)
  dinv = jnp.where(deg > 0, lax.rsqrt(deg), 0.0)
  dinv16 = jnp.broadcast_to(dinv[:, None], (NA, D1))
  dinv40 = jnp.broadcast_to(dinv[:, None], (NA, D2))
  zz16 = jnp.zeros((NPT, D1), jnp.float32)
  zz40 = jnp.zeros((NPT, D2), jnp.float32)

  hs1 = _stage_a(x, W1, dinv16)
  p1 = _pass16(hs1, src2d, dst2d, zz16)

  b1r = b1.reshape(1, D1)
  b2r = b2.reshape(1, D2)

  hs2 = _stage_b(p1, hs1, dinv16, dinv40, b1r, W2)
  p2 = _pass40(hs2, src2d, dst2d, zz40)
  out = _stage_c(p2, hs2, dinv40, b2r)
  return out[:N, :40]


# speedup vs baseline: 52.4206x; 1.1337x over previous
import functools

import jax
import jax.numpy as jnp
from jax import lax
from jax.experimental import pallas as pl
from jax.experimental.pallas import tpu as pltpu
from jax.experimental.pallas import tpu_sc as plsc

N = 10000
NDEG = 10240
EROWS = 2500
RPT = 79
NPT = N // 16
D1 = 16
D2 = 40
RBLK = 2000

_MESH = dict(core_axis_name="c", subcore_axis_name="s")
_SC_PARAMS = pltpu.CompilerParams(use_tc_tiling_on_sc=False)


def _tile_rows(wid):
  rb = wid * 78 + jnp.maximum(wid - 28, 0)
  nblk = 78 + (wid >= 28).astype(jnp.int32)
  return rb, nblk


def _make_deg():
  mesh = plsc.VectorSubcoreMesh(**_MESH)

  @functools.partial(
      pl.kernel,
      out_type=jax.ShapeDtypeStruct((2, NDEG), jnp.float32),
      mesh=mesh,
      compiler_params=_SC_PARAMS,
      scratch_types=[
          pltpu.VMEM((RPT, 128), jnp.int32),
          pltpu.VMEM((128,), jnp.float32),
          pltpu.VMEM((NDEG // 16,), jnp.float32),
          pltpu.VMEM_SHARED((NDEG,), jnp.float32),
      ],
  )
  def deg_kernel(dst_hbm, out_hbm, idx_v, ones_v, zero_v, acc):
    c = lax.axis_index("c")
    s = lax.axis_index("s")
    wid = c * 16 + s
    rb, nblk = _tile_rows(wid)
    npt = NDEG // 16
    one16 = jnp.full((16,), 1.0, jnp.float32)
    zero16 = jnp.zeros((16,), jnp.float32)
    for i in range(8):
      ones_v[pl.ds(i * 16, 16)] = one16

    def zbody(i, _):
      zero_v[pl.ds(i * 16, 16)] = zero16
      return 0

    lax.fori_loop(0, npt // 16, zbody, 0)
    pltpu.sync_copy(zero_v, acc.at[pl.ds(s * npt, npt)])
    pltpu.sync_copy(dst_hbm.at[pl.ds(rb, RPT)], idx_v)
    plsc.subcore_barrier()

    def body(j, _):
      pltpu.sync_copy(ones_v, acc.at[idx_v.at[j]], add=True)
      return 0

    lax.fori_loop(0, nblk, body, 0)
    plsc.subcore_barrier()
    pltpu.sync_copy(acc.at[pl.ds(s * npt, npt)],
                    out_hbm.at[c, pl.ds(s * npt, npt)])

  return deg_kernel


def _make_pass(d):
  mesh = plsc.VectorSubcoreMesh(**_MESH)

  @functools.partial(
      pl.kernel,
      out_type=jax.ShapeDtypeStruct((2, N, d), jnp.float32),
      mesh=mesh,
      compiler_params=_SC_PARAMS,
      scratch_types=[
          pltpu.VMEM((RPT, 128), jnp.int32),
          pltpu.VMEM((RPT, 128), jnp.int32),
          pltpu.VMEM((128, d), jnp.float32),
          pltpu.VMEM((128, d), jnp.float32),
          pltpu.VMEM_SHARED((N, d), jnp.float32),
          pltpu.VMEM_SHARED((N, d), jnp.float32),
          pltpu.SemaphoreType.DMA,
          pltpu.SemaphoreType.DMA,
      ],
  )
  def pass_kernel(hs_hbm, src_hbm, dst_hbm, zz_hbm, out_hbm,
                  sidx, didx, rows0, rows1, table, acc, sem0, sem1):
    c = lax.axis_index("c")
    s = lax.axis_index("s")
    wid = c * 16 + s
    rb, nblk = _tile_rows(wid)
    pltpu.sync_copy(hs_hbm.at[pl.ds(s * NPT, NPT)],
                    table.at[pl.ds(s * NPT, NPT)])
    pltpu.sync_copy(zz_hbm, acc.at[pl.ds(s * NPT, NPT)])
    pltpu.sync_copy(src_hbm.at[pl.ds(rb, RPT)], sidx)
    pltpu.sync_copy(dst_hbm.at[pl.ds(rb, RPT)], didx)
    plsc.subcore_barrier()

    pltpu.async_copy(table.at[sidx.at[0]], rows0, sem0)

    def body(i, _):
      b0 = 2 * i
      b1 = 2 * i + 1
      pltpu.async_copy(table.at[sidx.at[b1]], rows1, sem1)
      pltpu.make_async_copy(table.at[sidx.at[b0]], rows0, sem0).wait()
      pltpu.sync_copy(rows0, acc.at[didx.at[b0]], add=True)

      @pl.when(b0 + 2 < nblk)
      def _():
        pltpu.async_copy(table.at[sidx.at[b0 + 2]], rows0, sem0)

      pltpu.make_async_copy(table.at[sidx.at[b1]], rows1, sem1).wait()
      pltpu.sync_copy(rows1, acc.at[didx.at[b1]], add=True)
      return 0

    lax.fori_loop(0, 39, body, 0)

    @pl.when(nblk == RPT)
    def _():
      pltpu.make_async_copy(table.at[sidx.at[RPT - 1]], rows0, sem0).wait()
      pltpu.sync_copy(rows0, acc.at[didx.at[RPT - 1]], add=True)

    plsc.subcore_barrier()
    pltpu.sync_copy(acc.at[pl.ds(s * NPT, NPT)],
                    out_hbm.at[c, pl.ds(s * NPT, NPT)])

  return pass_kernel


_deg_call = _make_deg()
_pass16 = _make_pass(D1)
_pass40 = _make_pass(D2)


def _stage_a_body(x_ref, w_ref, d_ref, o_ref):
  o_ref[:, :] = jnp.dot(x_ref[:, :], w_ref[:, :],
                        preferred_element_type=jnp.float32) * d_ref[:, :]


def _stage_a(x, w1, dinv_col):
  grid = (N // RBLK,)
  return pl.pallas_call(
      _stage_a_body,
      grid=grid,
      in_specs=[
          pl.BlockSpec((RBLK, 128), lambda i: (i, 0)),
          pl.BlockSpec((128, D1), lambda i: (0, 0)),
          pl.BlockSpec((RBLK, 1), lambda i: (i, 0)),
      ],
      out_specs=pl.BlockSpec((RBLK, D1), lambda i: (i, 0)),
      out_shape=jax.ShapeDtypeStruct((N, D1), jnp.float32),
  )(x, w1, dinv_col)


def _stage_b_body(p_ref, h_ref, d_ref, b1_ref, w2_ref, o_ref):
  dcol = d_ref[:, :]
  t = (p_ref[0] + p_ref[1] + h_ref[:, :]) * dcol + b1_ref[:, :]
  t = jnp.maximum(t, 0.0)
  o_ref[:, :] = jnp.dot(t, w2_ref[:, :],
                        preferred_element_type=jnp.float32) * dcol


def _stage_b(p1, hs1, dinv_col, b1r, w2):
  grid = (N // RBLK,)
  return pl.pallas_call(
      _stage_b_body,
      grid=grid,
      in_specs=[
          pl.BlockSpec((2, RBLK, D1), lambda i: (0, i, 0)),
          pl.BlockSpec((RBLK, D1), lambda i: (i, 0)),
          pl.BlockSpec((RBLK, 1), lambda i: (i, 0)),
          pl.BlockSpec((1, D1), lambda i: (0, 0)),
          pl.BlockSpec((D1, D2), lambda i: (0, 0)),
      ],
      out_specs=pl.BlockSpec((RBLK, D2), lambda i: (i, 0)),
      out_shape=jax.ShapeDtypeStruct((N, D2), jnp.float32),
  )(p1, hs1, dinv_col, b1r, w2)


def _stage_c_body(p_ref, h_ref, d_ref, b2_ref, o_ref):
  logits = ((p_ref[0] + p_ref[1] + h_ref[:, :]) * d_ref[:, :]
            + b2_ref[:, :])
  m = jnp.max(logits, axis=1, keepdims=True)
  e = jnp.exp(logits - m)
  ssum = jnp.sum(e, axis=1, keepdims=True)
  o_ref[:, :] = logits - m - jnp.log(ssum)


def _stage_c(p2, hs2, dinv_col, b2r):
  grid = (N // RBLK,)
  return pl.pallas_call(
      _stage_c_body,
      grid=grid,
      in_specs=[
          pl.BlockSpec((2, RBLK, D2), lambda i: (0, i, 0)),
          pl.BlockSpec((RBLK, D2), lambda i: (i, 0)),
          pl.BlockSpec((RBLK, 1), lambda i: (i, 0)),
          pl.BlockSpec((1, D2), lambda i: (0, 0)),
      ],
      out_specs=pl.BlockSpec((RBLK, D2), lambda i: (i, 0)),
      out_shape=jax.ShapeDtypeStruct((N, D2), jnp.float32),
  )(p2, hs2, dinv_col, b2r)


def kernel(x, edge_index, W1, b1, W2, b2):
  ei = edge_index.astype(jnp.int32).reshape(2, EROWS, 128)
  src2d = ei[0]
  dst2d = ei[1]

  deg_p = _deg_call(dst2d)
  deg = deg_p[0, :N] + deg_p[1, :N] + 1.0
  dinv_col = lax.rsqrt(deg).reshape(N, 1)
  zz16 = jnp.zeros((NPT, D1), jnp.float32)
  zz40 = jnp.zeros((NPT, D2), jnp.float32)

  hs1 = _stage_a(x, W1, dinv_col)
  p1 = _pass16(hs1, src2d, dst2d, zz16)

  b1r = b1.reshape(1, D1)
  b2r = b2.reshape(1, D2)

  hs2 = _stage_b(p1, hs1, dinv_col, b1r, W2)
  p2 = _pass40(hs2, src2d, dst2d, zz40)
  return _stage_c(p2, hs2, dinv_col, b2r)
